# baseline (speedup 1.0000x reference)
import functools

import jax
import jax.numpy as jnp
from jax import lax
from jax.experimental import pallas as pl
from jax.experimental.pallas import tpu as pltpu
from jax.experimental.pallas import tpu_sc as plsc

N = 10000
NPAD = 10240
D = 128
DE = 16
E = 320000
E2 = 2 * E

_ROWS_A = 512


def _score_body(x_ref, wt_ref, b_ref, srow_ref, scol_ref, y_ref, sflat_ref):
    i = pl.program_id(0)
    xb = x_ref[...]
    wt = wt_ref[...]
    z = jnp.dot(xb, wt, preferred_element_type=jnp.float32) + b_ref[0, 0]
    s = jnp.tanh(z[:, :1])
    grow = i * _ROWS_A + lax.broadcasted_iota(jnp.int32, (_ROWS_A, 1), 0)
    s = jnp.where(grow >= N, jnp.float32(-2.0), s)
    scol_ref[...] = s
    srow_ref[...] = s.reshape(1, _ROWS_A)
    sflat_ref[...] = s.reshape(_ROWS_A)
    y_ref[...] = xb * s


_score_call = pl.pallas_call(
    _score_body,
    grid=(NPAD // _ROWS_A,),
    in_specs=[
        pl.BlockSpec((_ROWS_A, D), lambda i: (i, 0)),
        pl.BlockSpec((D, 8), lambda i: (0, 0)),
        pl.BlockSpec((1, 1), lambda i: (0, 0)),
    ],
    out_specs=[
        pl.BlockSpec((1, _ROWS_A), lambda i: (0, i)),
        pl.BlockSpec((_ROWS_A, 1), lambda i: (i, 0)),
        pl.BlockSpec((_ROWS_A, D), lambda i: (i, 0)),
        pl.BlockSpec((_ROWS_A,), lambda i: (i,)),
    ],
    out_shape=[
        jax.ShapeDtypeStruct((1, NPAD), jnp.float32),
        jax.ShapeDtypeStruct((NPAD, 1), jnp.float32),
        jax.ShapeDtypeStruct((NPAD, D), jnp.float32),
        jax.ShapeDtypeStruct((N,), jnp.float32),
    ],
)

_BI = 2048
_BJ = 2048
_NJ = NPAD // _BJ


def _sub_fold(c):
    return jnp.sum(c.astype(jnp.int32).reshape(_BJ // 8, 8, _BI), axis=0)


def _rank_body(s_row_ref, s_col_ref, attr_ref, rank_ref, rank1d_ref,
               attr_out_ref, acc_ref):
    attr_out_ref[...] = attr_ref[...]
    i = pl.program_id(0)
    j = pl.program_id(1)
    si = s_row_ref[...]
    sj = s_col_ref[...]
    ifirst = i * _BI
    jfirst = j * _BJ
    low = jfirst + _BJ - 1 < ifirst
    high = jfirst > ifirst + _BI - 1

    @pl.when(j == 0)
    def _():
        acc_ref[...] = jnp.zeros_like(acc_ref)

    @pl.when(low)
    def _():
        acc_ref[...] += _sub_fold(sj >= si)

    @pl.when(high)
    def _():
        acc_ref[...] += _sub_fold(sj > si)

    @pl.when(jnp.logical_not(low | high))
    def _():
        gi = ifirst + lax.broadcasted_iota(jnp.int32, (_BJ, _BI), 1)
        gj = jfirst + lax.broadcasted_iota(jnp.int32, (_BJ, _BI), 0)
        acc_ref[...] += _sub_fold((sj > si) | ((sj == si) & (gj < gi)))

    @pl.when(j == _NJ - 1)
    def _():
        total = jnp.sum(acc_ref[...], axis=0, keepdims=True)
        rank_ref[...] = total
        rank1d_ref[...] = total.reshape(_BI)


_rank_call = pl.pallas_call(
    _rank_body,
    grid=(NPAD // _BI, _NJ),
    in_specs=[
        pl.BlockSpec((1, _BI), lambda i, j: (0, i)),
        pl.BlockSpec((_BJ, 1), lambda i, j: (j, 0)),
        pl.BlockSpec((E // 25, DE), lambda i, j: (i * _NJ + j, 0)),
    ],
    out_specs=[
        pl.BlockSpec((1, _BI), lambda i, j: (0, i)),
        pl.BlockSpec((_BI,), lambda i, j: (i,)),
        pl.BlockSpec((E // 25, DE), lambda i, j: (i * _NJ + j, 0)),
    ],
    out_shape=[
        jax.ShapeDtypeStruct((1, NPAD), jnp.int32),
        jax.ShapeDtypeStruct((N,), jnp.int32),
        jax.ShapeDtypeStruct((E, DE), jnp.float32),
    ],
    scratch_shapes=[pltpu.VMEM((8, _BI), jnp.int32)],
)

_NC = 2
_NS = 16
_NW = _NC * _NS
_ROWC = 128
_NCHUNK = N // _ROWC
_NTAIL = N - _NCHUNK * _ROWC
_CPT = (_NCHUNK + _NW - 1) // _NW
_TAIL_TILE = _NCHUNK % _NW
_ZC = 312

_EW_BIG = 79 * 128
_EW_SMALL = 78 * 128
_EBIG = 4


def _sc_body(y_hbm, rank_hbm, eidx_hbm, xp_hbm, eout_hbm, perm_hbm, bat_hbm,
             table_v, eidx_v, eout_v, rows_v, rk_v, vals_v,
             trows_v, trk_v, tvals_v, zero_v,
             sem_t, sem_e, sem_eo, sem_ld, sem_s):
    wid = lax.axis_index("s") * _NC + lax.axis_index("c")
    is_big = wid < _EBIG
    ecol = jnp.where(is_big, wid * _EW_BIG,
                     _EBIG * _EW_BIG + (wid - _EBIG) * _EW_SMALL)

    tbl_cp = pltpu.make_async_copy(rank_hbm, table_v, sem_t)
    tbl_cp.start()

    @pl.when(is_big)
    def _():
        pltpu.make_async_copy(
            eidx_hbm.at[:, pl.ds(ecol, _EW_BIG)], eidx_v, sem_e).start()

    @pl.when(jnp.logical_not(is_big))
    def _():
        pltpu.make_async_copy(
            eidx_hbm.at[:, pl.ds(ecol, _EW_SMALL)],
            eidx_v.at[:, pl.ds(0, _EW_SMALL)], sem_e).start()

    for t in range(_CPT):
        cid = wid + _NW * t

        @pl.when(cid < _NCHUNK)
        def _():
            r0 = cid * _ROWC
            pltpu.make_async_copy(
                y_hbm.at[pl.ds(r0, _ROWC)], rows_v.at[t], sem_ld).start()
            pltpu.make_async_copy(
                rank_hbm.at[pl.ds(r0, _ROWC)], rk_v.at[t], sem_ld).start()

    @pl.when(wid == _TAIL_TILE)
    def _():
        pltpu.make_async_copy(
            y_hbm.at[pl.ds(_NCHUNK * _ROWC, _NTAIL)], trows_v, sem_ld).start()
        pltpu.make_async_copy(
            rank_hbm.at[pl.ds(_NCHUNK * _ROWC, _NTAIL)], trk_v, sem_ld).start()

    tbl_cp.wait()

    @pl.when(is_big)
    def _():
        pltpu.make_async_copy(
            eidx_hbm.at[:, pl.ds(ecol, _EW_BIG)], eidx_v, sem_e).wait()

    @pl.when(jnp.logical_not(is_big))
    def _():
        pltpu.make_async_copy(
            eidx_hbm.at[:, pl.ds(ecol, _EW_SMALL)],
            eidx_v.at[:, pl.ds(0, _EW_SMALL)], sem_e).wait()

    for r in range(2):
        def _edge_step(t, carry, _r=r):
            base = t * 64
            for q in range(4):
                idx16 = eidx_v[_r, pl.ds(base + q * 16, 16)]
                eout_v[_r, pl.ds(base + q * 16, 16)] = plsc.load_gather(
                    table_v, [idx16])
            return carry

        lax.fori_loop(0, _EW_SMALL // 64, _edge_step, 0)

        @pl.when(is_big)
        def _(_r=r):
            def _tail_step(t, carry):
                base = _EW_SMALL + t * 64
                for q in range(4):
                    idx16 = eidx_v[_r, pl.ds(base + q * 16, 16)]
                    eout_v[_r, pl.ds(base + q * 16, 16)] = (
                        plsc.load_gather(table_v, [idx16]))
                return carry

            lax.fori_loop(0, (_EW_BIG - _EW_SMALL) // 64, _tail_step, 0)

    @pl.when(is_big)
    def _():
        pltpu.make_async_copy(
            eout_v, eout_hbm.at[:, pl.ds(ecol, _EW_BIG)], sem_eo).start()

    @pl.when(jnp.logical_not(is_big))
    def _():
        pltpu.make_async_copy(
            eout_v.at[:, pl.ds(0, _EW_SMALL)],
            eout_hbm.at[:, pl.ds(ecol, _EW_SMALL)], sem_eo).start()

    for t in range(_CPT):
        cid = wid + _NW * t

        @pl.when(cid < _NCHUNK)
        def _():
            r0 = cid * _ROWC
            pltpu.make_async_copy(
                y_hbm.at[pl.ds(r0, _ROWC)], rows_v.at[t], sem_ld).wait()
            pltpu.make_async_copy(
                rank_hbm.at[pl.ds(r0, _ROWC)], rk_v.at[t], sem_ld).wait()

    @pl.when(wid == _TAIL_TILE)
    def _():
        pltpu.make_async_copy(
            y_hbm.at[pl.ds(_NCHUNK * _ROWC, _NTAIL)], trows_v, sem_ld).wait()
        pltpu.make_async_copy(
            rank_hbm.at[pl.ds(_NCHUNK * _ROWC, _NTAIL)], trk_v, sem_ld).wait()

    for t in range(_CPT):
        cid = wid + _NW * t

        @pl.when(cid < _NCHUNK)
        def _():
            r0 = cid * _ROWC
            for q in range(_ROWC // 16):
                vals_v.at[t][pl.ds(q * 16, 16)] = (
                    r0 + q * 16 + lax.iota(jnp.int32, 16))
            pltpu.make_async_copy(
                rows_v.at[t], xp_hbm.at[rk_v.at[t]], sem_s).start()
            pltpu.make_async_copy(
                vals_v.at[t], perm_hbm.at[rk_v.at[t]], sem_s).start()

    @pl.when(wid == _TAIL_TILE)
    def _():
        tvals_v[...] = _NCHUNK * _ROWC + lax.iota(jnp.int32, 16)
        pltpu.make_async_copy(
            trows_v, xp_hbm.at[trk_v], sem_s).start()
        pltpu.make_async_copy(
            tvals_v, perm_hbm.at[trk_v], sem_s).start()

    for q in range(320 // 16):
        zero_v[pl.ds(q * 16, 16)] = jnp.zeros((16,), jnp.int32)
    zrep = pltpu.make_async_copy(
        zero_v.at[pl.ds(0, _ZC)], bat_hbm.at[pl.ds(wid * _ZC, _ZC)], sem_eo)
    zrep.start()

    @pl.when(wid == 0)
    def _():
        pltpu.make_async_copy(
            zero_v.at[pl.ds(0, 16)],
            bat_hbm.at[pl.ds(_NW * _ZC, N - _NW * _ZC)], sem_eo).start()

    for t in range(_CPT):
        cid = wid + _NW * t

        @pl.when(cid < _NCHUNK)
        def _():
            pltpu.make_async_copy(
                rows_v.at[t], xp_hbm.at[rk_v.at[t]], sem_s).wait()
            pltpu.make_async_copy(
                vals_v.at[t], perm_hbm.at[rk_v.at[t]], sem_s).wait()

    @pl.when(wid == _TAIL_TILE)
    def _():
        pltpu.make_async_copy(
            trows_v, xp_hbm.at[trk_v], sem_s).wait()
        pltpu.make_async_copy(
            tvals_v, perm_hbm.at[trk_v], sem_s).wait()

    zrep.wait()

    @pl.when(wid == 0)
    def _():
        pltpu.make_async_copy(
            zero_v.at[pl.ds(0, 16)],
            bat_hbm.at[pl.ds(_NW * _ZC, N - _NW * _ZC)], sem_eo).wait()

    @pl.when(is_big)
    def _():
        pltpu.make_async_copy(
            eout_v, eout_hbm.at[:, pl.ds(ecol, _EW_BIG)], sem_eo).wait()

    @pl.when(jnp.logical_not(is_big))
    def _():
        pltpu.make_async_copy(
            eout_v.at[:, pl.ds(0, _EW_SMALL)],
            eout_hbm.at[:, pl.ds(ecol, _EW_SMALL)], sem_eo).wait()


@functools.lru_cache(maxsize=1)
def _sc_scatter_call():
    mesh = plsc.VectorSubcoreMesh(core_axis_name="c", subcore_axis_name="s",
                                  num_cores=_NC, num_subcores=_NS)
    return pl.kernel(
        _sc_body,
        out_type=[
            jax.ShapeDtypeStruct((N, D), jnp.float32),
            jax.ShapeDtypeStruct((2, E), jnp.int32),
            jax.ShapeDtypeStruct((N,), jnp.int32),
            jax.ShapeDtypeStruct((N,), jnp.int32),
        ],
        mesh=mesh,
        compiler_params=pltpu.CompilerParams(needs_layout_passes=False),
        scratch_types=[
            pltpu.VMEM((N,), jnp.int32),
            pltpu.VMEM((2, _EW_BIG), jnp.int32),
            pltpu.VMEM((2, _EW_BIG), jnp.int32),
            pltpu.VMEM((_CPT, _ROWC, D), jnp.float32),
            pltpu.VMEM((_CPT, _ROWC), jnp.int32),
            pltpu.VMEM((_CPT, _ROWC), jnp.int32),
            pltpu.VMEM((_NTAIL, D), jnp.float32),
            pltpu.VMEM((_NTAIL,), jnp.int32),
            pltpu.VMEM((_NTAIL,), jnp.int32),
            pltpu.VMEM((320,), jnp.int32),
            pltpu.SemaphoreType.DMA,
            pltpu.SemaphoreType.DMA,
            pltpu.SemaphoreType.DMA,
            pltpu.SemaphoreType.DMA,
            pltpu.SemaphoreType.DMA,
        ],
    )


def kernel(x, edge_index, edge_attr, batch, W, b):
    wt8 = jnp.concatenate([W.T.astype(jnp.float32),
                           jnp.zeros((D, 7), jnp.float32)], axis=1)
    srow, scol, y, s_flat = _score_call(
        x, wt8, b.reshape(1, 1).astype(jnp.float32))
    _, rank, edge_attr_out = _rank_call(srow, scol, edge_attr)

    xp, edge_index_out, perm, batch_out = _sc_scatter_call()(
        y, rank, edge_index)
    return (xp, edge_index_out, edge_attr_out, batch_out, perm, s_flat)


# speedup vs baseline: 2.1601x; 2.1601x over previous
import functools

import jax
import jax.numpy as jnp
from jax import lax
from jax.experimental import pallas as pl
from jax.experimental.pallas import tpu as pltpu
from jax.experimental.pallas import tpu_sc as plsc

N = 10000
NPAD = 10240
D = 128
E = 320000
E2 = 2 * E

_ROWS_A = 512


def _score_body(x_ref, wt_ref, b_ref, srow_ref, scol_ref, y_ref, sflat_ref):
    i = pl.program_id(0)
    xb = x_ref[...]
    wt = wt_ref[...]
    z = jnp.dot(xb, wt, preferred_element_type=jnp.float32) + b_ref[0, 0]
    s = jnp.tanh(z[:, :1])
    grow = i * _ROWS_A + lax.broadcasted_iota(jnp.int32, (_ROWS_A, 1), 0)
    s = jnp.where(grow >= N, jnp.float32(-2.0), s)
    scol_ref[...] = s
    srow_ref[...] = s.reshape(1, _ROWS_A)
    sflat_ref[...] = s.reshape(_ROWS_A)
    y_ref[...] = xb * s


_score_call = pl.pallas_call(
    _score_body,
    grid=(NPAD // _ROWS_A,),
    in_specs=[
        pl.BlockSpec((_ROWS_A, D), lambda i: (i, 0)),
        pl.BlockSpec((D, 8), lambda i: (0, 0)),
        pl.BlockSpec((1, 1), lambda i: (0, 0)),
    ],
    out_specs=[
        pl.BlockSpec((1, _ROWS_A), lambda i: (0, i)),
        pl.BlockSpec((_ROWS_A, 1), lambda i: (i, 0)),
        pl.BlockSpec((_ROWS_A, D), lambda i: (i, 0)),
        pl.BlockSpec((_ROWS_A,), lambda i: (i,)),
    ],
    out_shape=[
        jax.ShapeDtypeStruct((1, NPAD), jnp.float32),
        jax.ShapeDtypeStruct((NPAD, 1), jnp.float32),
        jax.ShapeDtypeStruct((NPAD, D), jnp.float32),
        jax.ShapeDtypeStruct((N,), jnp.float32),
    ],
)

_BI = 2048
_BJ = 2048
_NJ = NPAD // _BJ


def _sub_fold(c):
    return jnp.sum(c.astype(jnp.int32).reshape(_BJ // 8, 8, _BI), axis=0)


def _rank_body(s_row_ref, s_col_ref, rank_ref, rank1d_ref, acc_ref):
    i = pl.program_id(0)
    j = pl.program_id(1)
    si = s_row_ref[...]
    sj = s_col_ref[...]
    ifirst = i * _BI
    jfirst = j * _BJ
    low = jfirst + _BJ - 1 < ifirst
    high = jfirst > ifirst + _BI - 1

    @pl.when(j == 0)
    def _():
        acc_ref[...] = jnp.zeros_like(acc_ref)

    @pl.when(low)
    def _():
        acc_ref[...] += _sub_fold(sj >= si)

    @pl.when(high)
    def _():
        acc_ref[...] += _sub_fold(sj > si)

    @pl.when(jnp.logical_not(low | high))
    def _():
        gi = ifirst + lax.broadcasted_iota(jnp.int32, (_BJ, _BI), 1)
        gj = jfirst + lax.broadcasted_iota(jnp.int32, (_BJ, _BI), 0)
        acc_ref[...] += _sub_fold((sj > si) | ((sj == si) & (gj < gi)))

    @pl.when(j == _NJ - 1)
    def _():
        total = jnp.sum(acc_ref[...], axis=0, keepdims=True)
        rank_ref[...] = total
        rank1d_ref[...] = total.reshape(_BI)


_rank_call = pl.pallas_call(
    _rank_body,
    grid=(NPAD // _BI, _NJ),
    in_specs=[
        pl.BlockSpec((1, _BI), lambda i, j: (0, i)),
        pl.BlockSpec((_BJ, 1), lambda i, j: (j, 0)),
    ],
    out_specs=[
        pl.BlockSpec((1, _BI), lambda i, j: (0, i)),
        pl.BlockSpec((_BI,), lambda i, j: (i,)),
    ],
    out_shape=[
        jax.ShapeDtypeStruct((1, NPAD), jnp.int32),
        jax.ShapeDtypeStruct((N,), jnp.int32),
    ],
    scratch_shapes=[pltpu.VMEM((8, _BI), jnp.int32)],
)

_NC = 2
_NS = 16
_NW = _NC * _NS
_ROWC = 128
_NCHUNK = N // _ROWC
_NTAIL = N - _NCHUNK * _ROWC
_CPT = (_NCHUNK + _NW - 1) // _NW
_TAIL_TILE = _NCHUNK % _NW
_ZC = 312

_EW_BIG = 79 * 128
_EW_SMALL = 78 * 128
_EBIG = 4


def _sc_body(y_hbm, rank_hbm, eidx_hbm, xp_hbm, eout_hbm, perm_hbm, bat_hbm,
             table_v, eidx_v, eout_v, rows_v, rk_v, vals_v,
             trows_v, trk_v, tvals_v, zero_v,
             sem_t, sem_e, sem_eo, sem_ld, sem_s):
    wid = lax.axis_index("s") * _NC + lax.axis_index("c")
    is_big = wid < _EBIG
    ecol = jnp.where(is_big, wid * _EW_BIG,
                     _EBIG * _EW_BIG + (wid - _EBIG) * _EW_SMALL)

    tbl_cp = pltpu.make_async_copy(rank_hbm, table_v, sem_t)
    tbl_cp.start()

    @pl.when(is_big)
    def _():
        pltpu.make_async_copy(
            eidx_hbm.at[:, pl.ds(ecol, _EW_BIG)], eidx_v, sem_e).start()

    @pl.when(jnp.logical_not(is_big))
    def _():
        pltpu.make_async_copy(
            eidx_hbm.at[:, pl.ds(ecol, _EW_SMALL)],
            eidx_v.at[:, pl.ds(0, _EW_SMALL)], sem_e).start()

    for t in range(_CPT):
        cid = wid + _NW * t

        @pl.when(cid < _NCHUNK)
        def _():
            r0 = cid * _ROWC
            pltpu.make_async_copy(
                y_hbm.at[pl.ds(r0, _ROWC)], rows_v.at[t], sem_ld).start()
            pltpu.make_async_copy(
                rank_hbm.at[pl.ds(r0, _ROWC)], rk_v.at[t], sem_ld).start()

    @pl.when(wid == _TAIL_TILE)
    def _():
        pltpu.make_async_copy(
            y_hbm.at[pl.ds(_NCHUNK * _ROWC, _NTAIL)], trows_v, sem_ld).start()
        pltpu.make_async_copy(
            rank_hbm.at[pl.ds(_NCHUNK * _ROWC, _NTAIL)], trk_v, sem_ld).start()

    tbl_cp.wait()

    @pl.when(is_big)
    def _():
        pltpu.make_async_copy(
            eidx_hbm.at[:, pl.ds(ecol, _EW_BIG)], eidx_v, sem_e).wait()

    @pl.when(jnp.logical_not(is_big))
    def _():
        pltpu.make_async_copy(
            eidx_hbm.at[:, pl.ds(ecol, _EW_SMALL)],
            eidx_v.at[:, pl.ds(0, _EW_SMALL)], sem_e).wait()

    for r in range(2):
        def _edge_step(t, carry, _r=r):
            base = t * 64
            for q in range(4):
                idx16 = eidx_v[_r, pl.ds(base + q * 16, 16)]
                eout_v[_r, pl.ds(base + q * 16, 16)] = plsc.load_gather(
                    table_v, [idx16])
            return carry

        lax.fori_loop(0, _EW_SMALL // 64, _edge_step, 0)

        @pl.when(is_big)
        def _(_r=r):
            def _tail_step(t, carry):
                base = _EW_SMALL + t * 64
                for q in range(4):
                    idx16 = eidx_v[_r, pl.ds(base + q * 16, 16)]
                    eout_v[_r, pl.ds(base + q * 16, 16)] = (
                        plsc.load_gather(table_v, [idx16]))
                return carry

            lax.fori_loop(0, (_EW_BIG - _EW_SMALL) // 64, _tail_step, 0)

    @pl.when(is_big)
    def _():
        pltpu.make_async_copy(
            eout_v, eout_hbm.at[:, pl.ds(ecol, _EW_BIG)], sem_eo).start()

    @pl.when(jnp.logical_not(is_big))
    def _():
        pltpu.make_async_copy(
            eout_v.at[:, pl.ds(0, _EW_SMALL)],
            eout_hbm.at[:, pl.ds(ecol, _EW_SMALL)], sem_eo).start()

    for t in range(_CPT):
        cid = wid + _NW * t

        @pl.when(cid < _NCHUNK)
        def _():
            r0 = cid * _ROWC
            pltpu.make_async_copy(
                y_hbm.at[pl.ds(r0, _ROWC)], rows_v.at[t], sem_ld).wait()
            pltpu.make_async_copy(
                rank_hbm.at[pl.ds(r0, _ROWC)], rk_v.at[t], sem_ld).wait()

    @pl.when(wid == _TAIL_TILE)
    def _():
        pltpu.make_async_copy(
            y_hbm.at[pl.ds(_NCHUNK * _ROWC, _NTAIL)], trows_v, sem_ld).wait()
        pltpu.make_async_copy(
            rank_hbm.at[pl.ds(_NCHUNK * _ROWC, _NTAIL)], trk_v, sem_ld).wait()

    for t in range(_CPT):
        cid = wid + _NW * t

        @pl.when(cid < _NCHUNK)
        def _():
            r0 = cid * _ROWC
            for q in range(_ROWC // 16):
                vals_v.at[t][pl.ds(q * 16, 16)] = (
                    r0 + q * 16 + lax.iota(jnp.int32, 16))
            pltpu.make_async_copy(
                rows_v.at[t], xp_hbm.at[rk_v.at[t]], sem_s).start()
            pltpu.make_async_copy(
                vals_v.at[t], perm_hbm.at[rk_v.at[t]], sem_s).start()

    @pl.when(wid == _TAIL_TILE)
    def _():
        tvals_v[...] = _NCHUNK * _ROWC + lax.iota(jnp.int32, 16)
        pltpu.make_async_copy(
            trows_v, xp_hbm.at[trk_v], sem_s).start()
        pltpu.make_async_copy(
            tvals_v, perm_hbm.at[trk_v], sem_s).start()

    for q in range(320 // 16):
        zero_v[pl.ds(q * 16, 16)] = jnp.zeros((16,), jnp.int32)
    zrep = pltpu.make_async_copy(
        zero_v.at[pl.ds(0, _ZC)], bat_hbm.at[pl.ds(wid * _ZC, _ZC)], sem_eo)
    zrep.start()

    @pl.when(wid == 0)
    def _():
        pltpu.make_async_copy(
            zero_v.at[pl.ds(0, 16)],
            bat_hbm.at[pl.ds(_NW * _ZC, N - _NW * _ZC)], sem_eo).start()

    for t in range(_CPT):
        cid = wid + _NW * t

        @pl.when(cid < _NCHUNK)
        def _():
            pltpu.make_async_copy(
                rows_v.at[t], xp_hbm.at[rk_v.at[t]], sem_s).wait()
            pltpu.make_async_copy(
                vals_v.at[t], perm_hbm.at[rk_v.at[t]], sem_s).wait()

    @pl.when(wid == _TAIL_TILE)
    def _():
        pltpu.make_async_copy(
            trows_v, xp_hbm.at[trk_v], sem_s).wait()
        pltpu.make_async_copy(
            tvals_v, perm_hbm.at[trk_v], sem_s).wait()

    zrep.wait()

    @pl.when(wid == 0)
    def _():
        pltpu.make_async_copy(
            zero_v.at[pl.ds(0, 16)],
            bat_hbm.at[pl.ds(_NW * _ZC, N - _NW * _ZC)], sem_eo).wait()

    @pl.when(is_big)
    def _():
        pltpu.make_async_copy(
            eout_v, eout_hbm.at[:, pl.ds(ecol, _EW_BIG)], sem_eo).wait()

    @pl.when(jnp.logical_not(is_big))
    def _():
        pltpu.make_async_copy(
            eout_v.at[:, pl.ds(0, _EW_SMALL)],
            eout_hbm.at[:, pl.ds(ecol, _EW_SMALL)], sem_eo).wait()


@functools.lru_cache(maxsize=1)
def _sc_scatter_call():
    mesh = plsc.VectorSubcoreMesh(core_axis_name="c", subcore_axis_name="s",
                                  num_cores=_NC, num_subcores=_NS)
    return pl.kernel(
        _sc_body,
        out_type=[
            jax.ShapeDtypeStruct((N, D), jnp.float32),
            jax.ShapeDtypeStruct((2, E), jnp.int32),
            jax.ShapeDtypeStruct((N,), jnp.int32),
            jax.ShapeDtypeStruct((N,), jnp.int32),
        ],
        mesh=mesh,
        compiler_params=pltpu.CompilerParams(needs_layout_passes=False),
        scratch_types=[
            pltpu.VMEM((N,), jnp.int32),
            pltpu.VMEM((2, _EW_BIG), jnp.int32),
            pltpu.VMEM((2, _EW_BIG), jnp.int32),
            pltpu.VMEM((_CPT, _ROWC, D), jnp.float32),
            pltpu.VMEM((_CPT, _ROWC), jnp.int32),
            pltpu.VMEM((_CPT, _ROWC), jnp.int32),
            pltpu.VMEM((_NTAIL, D), jnp.float32),
            pltpu.VMEM((_NTAIL,), jnp.int32),
            pltpu.VMEM((_NTAIL,), jnp.int32),
            pltpu.VMEM((320,), jnp.int32),
            pltpu.SemaphoreType.DMA,
            pltpu.SemaphoreType.DMA,
            pltpu.SemaphoreType.DMA,
            pltpu.SemaphoreType.DMA,
            pltpu.SemaphoreType.DMA,
        ],
    )


def kernel(x, edge_index, edge_attr, batch, W, b):
    wt8 = jnp.concatenate([W.T.astype(jnp.float32),
                           jnp.zeros((D, 7), jnp.float32)], axis=1)
    srow, scol, y, s_flat = _score_call(
        x, wt8, b.reshape(1, 1).astype(jnp.float32))
    _, rank = _rank_call(srow, scol)

    xp, edge_index_out, perm, batch_out = _sc_scatter_call()(
        y, rank, edge_index)
    return (xp, edge_index_out, edge_attr, batch_out, perm, s_flat)
